# Initial kernel scaffold; baseline (speedup 1.0000x reference)
import jax
import jax.numpy as jnp
from jax.experimental import pallas as pl


def kernel(x, edge_index, edge_attr, W_enc, b_enc, W_conv, b_conv, W_cluster, b_cluster, W_energy, b_energy):
    raise NotImplementedError("write your pallas kernel here")


# speedup vs baseline: 2.1888x; 2.1888x over previous
import functools

import jax
import jax.numpy as jnp
from jax import lax
from jax.experimental import pallas as pl
from jax.experimental.pallas import tpu as pltpu
from jax.experimental.pallas import tpu_sc as plsc

N = 10000
E = 320000
H = 128
D_PAD = 8
NUM_CLUSTERS = 3

NC = 2
NS = 16
LANES = 16
NW = NC * NS

B = 128
NCHUNK = E // B
ROWS_PER_ZTILE = 1000
NZTILES = N // ROWS_PER_ZTILE

_mesh = plsc.VectorSubcoreMesh(
    core_axis_name="c", subcore_axis_name="s", num_cores=NC, num_subcores=NS)


def _chunk_bounds(tid):
    base = NCHUNK // NW
    extra = NCHUNK - base * NW
    return base + jnp.where(tid < extra, 1, 0)


@functools.partial(
    pl.kernel,
    out_type=(
        jax.ShapeDtypeStruct((E,), jnp.float32),
        jax.ShapeDtypeStruct((NC * N,), jnp.float32),
    ),
    mesh=_mesh,
    compiler_params=pltpu.CompilerParams(needs_layout_passes=False),
    scratch_types=[
        pltpu.VMEM((B,), jnp.float32),
        pltpu.VMEM((B,), jnp.int32),
        pltpu.VMEM((B,), jnp.float32),
        pltpu.VMEM((ROWS_PER_ZTILE,), jnp.float32),
        pltpu.VMEM_SHARED((N,), jnp.float32),
    ],
)
def _sw_kernel(ea_hbm, dst_hbm, w_hbm, sw_hbm, ea_v, dst_v, w_v, zv, shared_sw):
    cc = lax.axis_index("c")
    s = lax.axis_index("s")
    tid = s * NC + cc

    @pl.when(s < NZTILES)
    def _():
        for j in range(ROWS_PER_ZTILE // LANES):
            zv[pl.ds(j * LANES, LANES)] = jnp.zeros((LANES,), jnp.float32)
        pltpu.sync_copy(zv, shared_sw.at[pl.ds(s * ROWS_PER_ZTILE, ROWS_PER_ZTILE)])

    plsc.subcore_barrier()

    def body(k, carry):
        off = (tid + k * NW) * B
        pltpu.sync_copy(ea_hbm.at[pl.ds(off, B)], ea_v)
        pltpu.sync_copy(dst_hbm.at[pl.ds(off, B)], dst_v)
        for j in range(B // LANES):
            a = ea_v[pl.ds(j * LANES, LANES)]
            t = a + 1e-06
            w_v[pl.ds(j * LANES, LANES)] = 1.0 / (t * t)
        pltpu.sync_copy(w_v, w_hbm.at[pl.ds(off, B)])
        pltpu.sync_copy(w_v, shared_sw.at[dst_v], add=True)
        return carry

    lax.fori_loop(0, _chunk_bounds(tid), body, 0)

    plsc.subcore_barrier()

    @pl.when(s < NZTILES)
    def _():
        r0 = s * ROWS_PER_ZTILE
        pltpu.sync_copy(shared_sw.at[pl.ds(r0, ROWS_PER_ZTILE)], zv)
        pltpu.sync_copy(zv, sw_hbm.at[pl.ds(cc * N + r0, ROWS_PER_ZTILE)])


_ZROWS = 8


@functools.partial(
    pl.kernel,
    out_type=jax.ShapeDtypeStruct((NC, N, H), jnp.float32),
    mesh=_mesh,
    compiler_params=pltpu.CompilerParams(needs_layout_passes=False),
    scratch_types=[
        pltpu.VMEM((B,), jnp.int32),
        pltpu.VMEM((B,), jnp.int32),
        pltpu.VMEM((B,), jnp.float32),
        pltpu.VMEM((B, H), jnp.float32),
        pltpu.VMEM((_ZROWS, H), jnp.float32),
        pltpu.VMEM((ROWS_PER_ZTILE // 5, H), jnp.float32),
        pltpu.VMEM_SHARED((N, H), jnp.float32),
    ],
)
def _spmm_kernel(h_hbm, src_hbm, dst_hbm, w_hbm, out_hbm,
                 src_v, dst_v, w_v, rows, zbuf, obuf, shared_a):
    cc = lax.axis_index("c")
    s = lax.axis_index("s")
    tid = s * NC + cc

    @pl.when(s < NZTILES)
    def _():
        for r in range(_ZROWS):
            for j in range(H // LANES):
                zbuf[r, pl.ds(j * LANES, LANES)] = jnp.zeros((LANES,), jnp.float32)
        r0 = s * ROWS_PER_ZTILE

        def zbody(k, carry):
            pltpu.sync_copy(zbuf, shared_a.at[pl.ds(r0 + k * _ZROWS, _ZROWS), :])
            return carry

        lax.fori_loop(0, ROWS_PER_ZTILE // _ZROWS, zbody, 0)

    plsc.subcore_barrier()

    cols0 = lax.iota(jnp.int32, LANES)

    def body(k, carry):
        off = (tid + k * NW) * B
        pltpu.sync_copy(src_hbm.at[pl.ds(off, B)], src_v)
        pltpu.sync_copy(dst_hbm.at[pl.ds(off, B)], dst_v)
        pltpu.sync_copy(w_hbm.at[pl.ds(off, B)], w_v)
        pltpu.sync_copy(h_hbm.at[src_v], rows)

        def scale(i, c2):
            ri = jnp.full((LANES,), i, jnp.int32)
            wv = plsc.load_gather(w_v, [ri])
            for j in range(H // LANES):
                cj = cols0 + (j * LANES)
                v = plsc.load_gather(rows, [ri, cj])
                plsc.store_scatter(rows, [ri, cj], v * wv)
            return c2

        lax.fori_loop(0, B, scale, 0)
        pltpu.sync_copy(rows, shared_a.at[dst_v], add=True)
        return carry

    lax.fori_loop(0, _chunk_bounds(tid), body, 0)

    plsc.subcore_barrier()

    @pl.when(s < NZTILES)
    def _():
        piece = ROWS_PER_ZTILE // 5
        for q in range(5):
            r0 = s * ROWS_PER_ZTILE + q * piece
            pltpu.sync_copy(shared_a.at[pl.ds(r0, piece), :], obuf)
            pltpu.sync_copy(obuf, out_hbm.at[cc, pl.ds(r0, piece), :])


def _enc_body(x_ref, w_ref, b_ref, o_ref):
    o_ref[...] = jnp.dot(x_ref[...], w_ref[...],
                         preferred_element_type=jnp.float32,
                         precision=lax.Precision.HIGHEST) + b_ref[...]


def _layer_body(a_ref, w_ref, sw_ref, b_ref, o_ref):
    acc = a_ref[0] + a_ref[1]
    y = jnp.dot(acc, w_ref[...], preferred_element_type=jnp.float32,
                precision=lax.Precision.HIGHEST)
    y = y + sw_ref[...] * b_ref[...]
    o_ref[...] = jnp.maximum(y, 0.0)


def _tail_body(h_ref, wc_ref, bc_ref, we_ref, be_ref, p_ref, e_ref):
    h = h_ref[...]
    logits = jnp.dot(h, wc_ref[...], preferred_element_type=jnp.float32,
                     precision=lax.Precision.HIGHEST) + bc_ref[...]
    m = jnp.max(logits, axis=-1, keepdims=True)
    ex = jnp.exp(logits - m)
    p_ref[...] = ex / jnp.sum(ex, axis=-1, keepdims=True)
    mean = jnp.sum(h, axis=0, keepdims=True) * (1.0 / N)
    e_ref[...] = jnp.dot(mean, we_ref[...], preferred_element_type=jnp.float32,
                         precision=lax.Precision.HIGHEST) + be_ref[...]


def kernel(x, edge_index, edge_attr, W_enc, b_enc, W_conv, b_conv,
           W_cluster, b_cluster, W_energy, b_energy):
    src = edge_index[0]
    dst = edge_index[1]

    w, sw_part = _sw_kernel(edge_attr, dst)
    sw_part = sw_part.reshape(NC, N)
    swm = (sw_part[0] + sw_part[1]).reshape(N, 1)

    x_pad = jnp.pad(x, ((0, 0), (0, D_PAD - x.shape[1])))
    we_pad = jnp.pad(W_enc, ((0, D_PAD - W_enc.shape[0]), (0, 0)))
    h = pl.pallas_call(
        _enc_body,
        out_shape=jax.ShapeDtypeStruct((N, H), jnp.float32),
    )(x_pad, we_pad, b_enc.reshape(1, H))

    layer = pl.pallas_call(
        _layer_body,
        out_shape=jax.ShapeDtypeStruct((N, H), jnp.float32),
    )
    for l in range(W_conv.shape[0]):
        a_part = _spmm_kernel(h, src, dst, w)
        h = layer(a_part, W_conv[l], swm, b_conv[l].reshape(1, H))

    wc_pad = jnp.pad(W_cluster, ((0, 0), (0, H - NUM_CLUSTERS)))
    bc_pad = jnp.pad(b_cluster.reshape(1, NUM_CLUSTERS),
                     ((0, 0), (0, H - NUM_CLUSTERS)),
                     constant_values=-jnp.inf)
    we2_pad = jnp.pad(W_energy, ((0, 0), (0, H - 1)))
    be_pad = jnp.pad(b_energy.reshape(1, 1), ((0, 0), (0, H - 1)))
    probs_pad, energy_pad = pl.pallas_call(
        _tail_body,
        out_shape=(
            jax.ShapeDtypeStruct((N, H), jnp.float32),
            jax.ShapeDtypeStruct((1, H), jnp.float32),
        ),
    )(h, wc_pad, bc_pad, we2_pad, be_pad)

    cluster_probs = probs_pad[:, :NUM_CLUSTERS]
    energy = energy_pad[:, :1]
    stable = energy < -1.0
    return (cluster_probs, energy, stable)
